# baseline (speedup 1.0000x reference)
import functools

import jax
import jax.numpy as jnp
from jax import lax
from jax.experimental import pallas as pl
from jax.experimental.pallas import tpu as pltpu
from jax.experimental.pallas import tpu_sc as plsc

_PAD = 1
_TBL = 1024


def _sc_fused(emb, w_row, b, indices2d):
    info = plsc.get_sparse_core_info()
    nc, ns, lanes = info.num_cores, info.num_subcores, info.num_lanes
    nw = nc * ns
    batch = indices2d.shape[1]
    bpw = batch // nw
    vocab, dim = emb.shape
    rpt = _TBL // ns
    grp = rpt // lanes
    max_base = vocab - rpt
    ebuf = 2 * rpt
    mesh = plsc.VectorSubcoreMesh(core_axis_name="c", subcore_axis_name="s")

    @functools.partial(
        pl.kernel,
        out_type=jax.ShapeDtypeStruct((batch,), jnp.float32),
        mesh=mesh,
        scratch_types=[
            pltpu.VMEM((ebuf, dim), jnp.float32),
            pltpu.VMEM((dim + lanes,), jnp.float32),
            pltpu.VMEM((lanes,), jnp.float32),
            pltpu.VMEM((rpt,), jnp.float32),
            pltpu.VMEM((_TBL,), jnp.float32),
            pltpu.VMEM((bpw,), jnp.int32),
            pltpu.VMEM((bpw,), jnp.float32),
            pltpu.VMEM_SHARED((_TBL,), jnp.float32),
            pltpu.SemaphoreType.DMA,
        ],
        compiler_params=pltpu.CompilerParams(needs_layout_passes=False),
    )
    def k(emb_hbm, w_hbm, b_hbm, idx_hbm, out_hbm,
          emb_v, w_v, b_v, tacc_v, table_v, idx_v, out_v, table_sh, sem):
        c = lax.axis_index("c")
        s = lax.axis_index("s")
        wid = s * nc + c
        obase = wid * bpw
        idx_cp = pltpu.async_copy(idx_hbm.at[0, pl.ds(obase, bpw)], idx_v, sem)

        row0 = s * rpt
        ebase = lax.min(row0, max_base)
        loff = row0 - ebase
        pltpu.sync_copy(emb_hbm.at[pl.ds(ebase, rpt), :], emb_v.at[pl.ds(0, rpt), :])
        pltpu.sync_copy(w_hbm.at[0], w_v.at[pl.ds(0, dim)])
        pltpu.sync_copy(b_hbm, b_v.at[pl.ds(0, 1)])
        lane = lax.broadcasted_iota(jnp.int32, (lanes,), 0)

        def dbody(d, accs):
            wd = w_v[pl.ds(d, lanes)][0]
            col = jnp.full((lanes,), d, jnp.int32)
            return tuple(
                accs[g] + plsc.load_gather(emb_v, [loff + g * lanes + lane, col]) * wd
                for g in range(grp)
            )

        zero = jnp.zeros((lanes,), jnp.float32)
        accs = lax.fori_loop(0, dim, dbody, (zero,) * grp)
        bias = b_v[...][0]
        for g in range(grp):
            rows = row0 + g * lanes + lane
            tacc_v[pl.ds(g * lanes, lanes)] = (
                jnp.where(rows == _PAD, 0.0, accs[g]) + bias
            )

        pltpu.sync_copy(tacc_v, table_sh.at[pl.ds(row0, rpt)])
        plsc.subcore_barrier()
        pltpu.sync_copy(table_sh, table_v)

        idx_cp.wait()
        for j in range(bpw // lanes):
            iv = idx_v[pl.ds(j * lanes, lanes)]
            out_v[pl.ds(j * lanes, lanes)] = plsc.load_gather(table_v, [iv])
        pltpu.sync_copy(out_v, out_hbm.at[pl.ds(obase, bpw)])

    return k(emb, w_row, b, indices2d)


def kernel(indices, emb, W, b):
    idx2d = indices.reshape(indices.shape[0], indices.shape[1])
    w_row = W.reshape(1, W.shape[0])
    return _sc_fused(emb, w_row, b, idx2d)[:, None]


# speedup vs baseline: 1.1840x; 1.1840x over previous
import functools

import jax
import jax.numpy as jnp
from jax import lax
from jax.experimental import pallas as pl
from jax.experimental.pallas import tpu as pltpu
from jax.experimental.pallas import tpu_sc as plsc

_VOCAB = 1000
_TBL_PAD = 1024
_PAD = 1


def _table_body(emb_ref, w_ref, b_ref, out_ref):
    t = lax.dot_general(
        w_ref[...], emb_ref[...], (((0,), (1,)), ((), ())),
        preferred_element_type=jnp.float32,
    )
    col = lax.broadcasted_iota(jnp.int32, t.shape, 1)
    out_ref[...] = jnp.where(col == _PAD, 0.0, t) + b_ref[...]


def _build_table(emb, w, b2):
    return pl.pallas_call(
        _table_body,
        out_shape=jax.ShapeDtypeStruct((1, _VOCAB), jnp.float32),
    )(emb, w, b2)


def _sc_lookup(table_row, idx):
    info = plsc.get_sparse_core_info()
    nw = info.num_cores * info.num_subcores
    lanes = info.num_lanes
    batch = idx.shape[0]
    bpw = batch // nw
    mesh = plsc.VectorSubcoreMesh(core_axis_name="c", subcore_axis_name="s")

    @functools.partial(
        pl.kernel,
        out_type=jax.ShapeDtypeStruct((batch,), jnp.float32),
        mesh=mesh,
        scratch_types=[
            pltpu.VMEM((_TBL_PAD,), jnp.float32),
            pltpu.VMEM((bpw,), jnp.int32),
            pltpu.VMEM((bpw,), jnp.float32),
            pltpu.SemaphoreType.DMA,
            pltpu.SemaphoreType.DMA,
        ],
        compiler_params=pltpu.CompilerParams(needs_layout_passes=False),
    )
    def k(table_hbm, idx_hbm, out_hbm, table_v, idx_v, out_v, sem_t, sem_i):
        wid = lax.axis_index("s") * info.num_cores + lax.axis_index("c")
        base = wid * bpw
        tbl_cp = pltpu.async_copy(table_hbm.at[0], table_v.at[pl.ds(0, _VOCAB)], sem_t)
        idx_cp = pltpu.async_copy(idx_hbm.at[pl.ds(base, bpw)], idx_v, sem_i)
        tbl_cp.wait()
        idx_cp.wait()
        for j in range(bpw // lanes):
            iv = idx_v[pl.ds(j * lanes, lanes)]
            out_v[pl.ds(j * lanes, lanes)] = plsc.load_gather(table_v, [iv])
        pltpu.sync_copy(out_v, out_hbm.at[pl.ds(base, bpw)])

    return k(table_row, idx)


def kernel(indices, emb, W, b):
    idx0 = indices[0, :, 0]
    table = _build_table(emb, W, b.reshape(1, 1))
    return _sc_lookup(table, idx0)[:, None]


# speedup vs baseline: 1.1913x; 1.0061x over previous
import functools

import jax
import jax.numpy as jnp
from jax import lax
from jax.experimental import pallas as pl
from jax.experimental.pallas import tpu as pltpu
from jax.experimental.pallas import tpu_sc as plsc

_VOCAB = 1000
_TBL_PAD = 1024
_PAD = 1


def _table_body(emb_ref, w_ref, b_ref, out_ref):
    t = lax.dot_general(
        w_ref[...], emb_ref[...], (((0,), (1,)), ((), ())),
        preferred_element_type=jnp.float32,
    )
    col = lax.broadcasted_iota(jnp.int32, t.shape, 1)
    out_ref[...] = jnp.where(col == _PAD, 0.0, t) + b_ref[...]


def _build_table(emb, w, b2):
    return pl.pallas_call(
        _table_body,
        out_shape=jax.ShapeDtypeStruct((1, _VOCAB), jnp.float32),
    )(emb, w, b2)


def _sc_lookup(table_row, idx):
    info = plsc.get_sparse_core_info()
    nw = info.num_cores * info.num_subcores
    lanes = info.num_lanes
    batch = idx.shape[0]
    bpw = batch // nw
    mesh = plsc.VectorSubcoreMesh(core_axis_name="c", subcore_axis_name="s")

    @functools.partial(
        pl.kernel,
        out_type=jax.ShapeDtypeStruct((batch,), jnp.float32),
        mesh=mesh,
        scratch_types=[
            pltpu.VMEM((_TBL_PAD,), jnp.float32),
            pltpu.VMEM((bpw,), jnp.int32),
            pltpu.VMEM((bpw,), jnp.float32),
            pltpu.SemaphoreType.DMA,
            pltpu.SemaphoreType.DMA,
        ],
        compiler_params=pltpu.CompilerParams(needs_layout_passes=False),
    )
    def k(table_hbm, idx_hbm, out_hbm, table_v, idx_v, out_v, sem_t, sem_i):
        wid = lax.axis_index("s") * info.num_cores + lax.axis_index("c")
        base = wid * bpw
        tbl_cp = pltpu.async_copy(table_hbm.at[0], table_v.at[pl.ds(0, _VOCAB)], sem_t)
        idx_cp = pltpu.async_copy(idx_hbm.at[pl.ds(base, bpw)], idx_v, sem_i)
        tbl_cp.wait()
        idx_cp.wait()
        for j in range(bpw // lanes):
            iv = idx_v[pl.ds(j * lanes, lanes)]
            out_v[pl.ds(j * lanes, lanes)] = plsc.load_gather(table_v, [iv])
        pltpu.sync_copy(out_v, out_hbm.at[pl.ds(base, bpw)])

    return k(table_row, idx)


def kernel(indices, emb, W, b):
    idx0 = indices[0, :, 0].astype(jnp.int32)
    table = _build_table(emb, W, b.reshape(1, 1))
    return _sc_lookup(table, idx0)[:, None]
